# baseline (speedup 1.0000x reference)
import functools

import jax
import jax.numpy as jnp
from jax import lax
from jax.experimental import pallas as pl
from jax.experimental.pallas import tpu as pltpu
from jax.experimental.pallas import tpu_sc as plsc

BATCH = 16384
EMBED_DIM = 64

_info = plsc.get_sparse_core_info()
_NC, _NS = _info.num_cores, _info.num_subcores
_NW = _NC * _NS
_BPW = BATCH // _NW
_RING = 32


def _make_kernel():
    mesh = plsc.VectorSubcoreMesh(core_axis_name="c", subcore_axis_name="s")

    @functools.partial(
        pl.kernel,
        mesh=mesh,
        out_type=(
            jax.ShapeDtypeStruct((BATCH, EMBED_DIM), jnp.float32),
            jax.ShapeDtypeStruct((BATCH, EMBED_DIM), jnp.float32),
        ),
        scratch_types=[
            pltpu.VMEM((_BPW,), jnp.int32),
            pltpu.VMEM((_BPW,), jnp.int32),
            pltpu.SemaphoreType.DMA,
            pltpu.SemaphoreType.DMA,
        ],
    )
    def emb_kernel(e1_hbm, rel_hbm, tab_e_hbm, tab_r_hbm, out_e_hbm,
                   out_r_hbm, idx_e, idx_r, sem_e, sem_r):
        wid = lax.axis_index("s") * _NC + lax.axis_index("c")
        base = wid * _BPW
        pltpu.sync_copy(e1_hbm.at[pl.ds(base, _BPW)], idx_e)
        pltpu.sync_copy(rel_hbm.at[pl.ds(base, _BPW)], idx_r)

        def wait_one(tab, out, sem):
            pltpu.make_async_copy(
                tab.at[pl.ds(0, 1)], out.at[pl.ds(base, 1)], sem).wait()

        L = 16

        def body(c, _):
            j0 = c * L
            v_e = idx_e[pl.ds(j0, L)]
            v_r = idx_r[pl.ds(j0, L)]
            for lane in range(L):
                pltpu.make_async_copy(
                    tab_e_hbm.at[pl.ds(v_e[lane], 1)],
                    out_e_hbm.at[pl.ds(base + j0 + lane, 1)], sem_e).start()
                pltpu.make_async_copy(
                    tab_r_hbm.at[pl.ds(v_r[lane], 1)],
                    out_r_hbm.at[pl.ds(base + j0 + lane, 1)], sem_r).start()

            @pl.when(c >= 1)
            def _wait():
                for _i in range(L):
                    wait_one(tab_e_hbm, out_e_hbm, sem_e)
                    wait_one(tab_r_hbm, out_r_hbm, sem_r)

            return None

        lax.fori_loop(0, _BPW // L, body, None, unroll=False)

        for _i in range(L):
            wait_one(tab_e_hbm, out_e_hbm, sem_e)
            wait_one(tab_r_hbm, out_r_hbm, sem_r)

    return emb_kernel


_emb_kernel = _make_kernel()


def kernel(e1, rel, emb_e_weight, emb_rel_weight):
    e1_flat = e1.reshape(BATCH)
    rel_flat = rel.reshape(BATCH)
    return _emb_kernel(e1_flat, rel_flat, emb_e_weight, emb_rel_weight)


# speedup vs baseline: 2.5077x; 2.5077x over previous
import functools

import jax
import jax.numpy as jnp
from jax import lax
from jax.experimental import pallas as pl
from jax.experimental.pallas import tpu as pltpu
from jax.experimental.pallas import tpu_sc as plsc

BATCH = 16384
EMBED_DIM = 64

_info = plsc.get_sparse_core_info()
_NC, _NS = _info.num_cores, _info.num_subcores
_NW = _NC * _NS
_BPW = BATCH // _NW
_CH = 32
_NCHUNK = _BPW // _CH
_LANES = 16


def _make_kernel():
    mesh = plsc.VectorSubcoreMesh(core_axis_name="c", subcore_axis_name="s")

    @functools.partial(
        pl.kernel,
        mesh=mesh,
        out_type=(
            jax.ShapeDtypeStruct((BATCH, EMBED_DIM), jnp.float32),
            jax.ShapeDtypeStruct((BATCH, EMBED_DIM), jnp.float32),
        ),
        scratch_types=[
            pltpu.VMEM((_BPW,), jnp.int32),
            pltpu.VMEM((_BPW,), jnp.int32),
            pltpu.VMEM((_CH, 8, EMBED_DIM), jnp.float32),
            pltpu.VMEM((_CH, EMBED_DIM), jnp.float32),
            pltpu.SemaphoreType.DMA,
        ],
    )
    def emb_kernel(e1_hbm, rel_hbm, tab_e_hbm, tab_r_hbm, out_e_hbm,
                   out_r_hbm, idx_e, idx_r, blocks, stage, sem):
        wid = lax.axis_index("s") * _NC + lax.axis_index("c")
        base = wid * _BPW
        pltpu.sync_copy(e1_hbm.at[pl.ds(base, _BPW)], idx_e)
        pltpu.sync_copy(rel_hbm.at[pl.ds(base, _BPW)], idx_r)

        def lookup_table(tab_hbm, idx, out_hbm):
            def chunk_body(k, carry):
                j0 = k * _CH
                for g in range(_CH // _LANES):
                    v_blk = idx[pl.ds(j0 + g * _LANES, _LANES)] >> 3
                    for lane in range(_LANES):
                        j = g * _LANES + lane
                        pltpu.make_async_copy(
                            tab_hbm.at[pl.ds(v_blk[lane], 1)],
                            blocks.at[pl.ds(j, 1)], sem).start()
                pltpu.make_async_copy(
                    tab_hbm.at[pl.ds(0, _CH)], blocks, sem).wait()
                for g in range(_CH // _LANES):
                    v_sub = idx[pl.ds(j0 + g * _LANES, _LANES)] & 7
                    for lane in range(_LANES):
                        j = g * _LANES + lane
                        r = v_sub[lane]
                        for c in range(0, EMBED_DIM, _LANES):
                            stage[j, pl.ds(c, _LANES)] = (
                                blocks[j, r, pl.ds(c, _LANES)])
                pltpu.sync_copy(stage, out_hbm.at[pl.ds(base + j0, _CH)])
                return carry

            lax.fori_loop(0, _NCHUNK, chunk_body, None, unroll=False)

        lookup_table(tab_e_hbm, idx_e, out_e_hbm)
        lookup_table(tab_r_hbm, idx_r, out_r_hbm)

    return emb_kernel


_emb_kernel = _make_kernel()


def kernel(e1, rel, emb_e_weight, emb_rel_weight):
    e1_flat = e1.reshape(BATCH)
    rel_flat = rel.reshape(BATCH)
    tab_e = emb_e_weight.reshape(-1, 8, EMBED_DIM)
    tab_r = emb_rel_weight.reshape(-1, 8, EMBED_DIM)
    return _emb_kernel(e1_flat, rel_flat, tab_e, tab_r)


# speedup vs baseline: 2.5723x; 1.0258x over previous
import functools

import jax
import jax.numpy as jnp
from jax import lax
from jax.experimental import pallas as pl
from jax.experimental.pallas import tpu as pltpu
from jax.experimental.pallas import tpu_sc as plsc

BATCH = 16384
EMBED_DIM = 64

_info = plsc.get_sparse_core_info()
_NC, _NS = _info.num_cores, _info.num_subcores
_NW = _NC * _NS
_BPW = BATCH // _NW
_CH = 64
_NCHUNK = _BPW // _CH
_LANES = 16


def _make_kernel():
    mesh = plsc.VectorSubcoreMesh(core_axis_name="c", subcore_axis_name="s")

    @functools.partial(
        pl.kernel,
        mesh=mesh,
        out_type=(
            jax.ShapeDtypeStruct((BATCH, EMBED_DIM), jnp.float32),
            jax.ShapeDtypeStruct((BATCH, EMBED_DIM), jnp.float32),
        ),
        scratch_types=[
            pltpu.VMEM((_BPW,), jnp.int32),
            pltpu.VMEM((_BPW,), jnp.int32),
            pltpu.VMEM((_CH, 8, EMBED_DIM), jnp.float32),
            pltpu.VMEM((_CH, EMBED_DIM), jnp.float32),
            pltpu.SemaphoreType.DMA,
        ],
    )
    def emb_kernel(e1_hbm, rel_hbm, tab_e_hbm, tab_r_hbm, out_e_hbm,
                   out_r_hbm, idx_e, idx_r, blocks, stage, sem):
        wid = lax.axis_index("s") * _NC + lax.axis_index("c")
        base = wid * _BPW
        pltpu.sync_copy(e1_hbm.at[pl.ds(base, _BPW)], idx_e)
        pltpu.sync_copy(rel_hbm.at[pl.ds(base, _BPW)], idx_r)

        def lookup_table(tab_hbm, idx, out_hbm):
            def chunk_body(k, carry):
                j0 = k * _CH
                for g in range(_CH // _LANES):
                    v_blk = idx[pl.ds(j0 + g * _LANES, _LANES)] >> 3
                    for lane in range(_LANES):
                        j = g * _LANES + lane
                        pltpu.make_async_copy(
                            tab_hbm.at[pl.ds(v_blk[lane], 1)],
                            blocks.at[pl.ds(j, 1)], sem).start()
                pltpu.make_async_copy(
                    tab_hbm.at[pl.ds(0, _CH)], blocks, sem).wait()
                for g in range(_CH // _LANES):
                    v_sub = idx[pl.ds(j0 + g * _LANES, _LANES)] & 7
                    for lane in range(_LANES):
                        j = g * _LANES + lane
                        r = v_sub[lane]
                        for c in range(0, EMBED_DIM, _LANES):
                            stage[j, pl.ds(c, _LANES)] = (
                                blocks[j, r, pl.ds(c, _LANES)])
                pltpu.sync_copy(stage, out_hbm.at[pl.ds(base + j0, _CH)])
                return carry

            lax.fori_loop(0, _NCHUNK, chunk_body, None, unroll=False)

        lookup_table(tab_e_hbm, idx_e, out_e_hbm)
        lookup_table(tab_r_hbm, idx_r, out_r_hbm)

    return emb_kernel


_emb_kernel = _make_kernel()


def kernel(e1, rel, emb_e_weight, emb_rel_weight):
    e1_flat = e1.reshape(BATCH)
    rel_flat = rel.reshape(BATCH)
    tab_e = emb_e_weight.reshape(-1, 8, EMBED_DIM)
    tab_r = emb_rel_weight.reshape(-1, 8, EMBED_DIM)
    return _emb_kernel(e1_flat, rel_flat, tab_e, tab_r)


# speedup vs baseline: 3.1776x; 1.2353x over previous
import functools

import jax
import jax.numpy as jnp
from jax import lax
from jax.experimental import pallas as pl
from jax.experimental.pallas import tpu as pltpu
from jax.experimental.pallas import tpu_sc as plsc

BATCH = 16384
EMBED_DIM = 64

_info = plsc.get_sparse_core_info()
_NC, _NS = _info.num_cores, _info.num_subcores
_NW = _NC * _NS
_BPW = BATCH // _NW
_CH = 64
_NCHUNK = _BPW // _CH
_LANES = 16


def _make_kernel():
    mesh = plsc.VectorSubcoreMesh(core_axis_name="c", subcore_axis_name="s")

    @functools.partial(
        pl.kernel,
        mesh=mesh,
        out_type=(
            jax.ShapeDtypeStruct((BATCH, EMBED_DIM), jnp.float32),
            jax.ShapeDtypeStruct((BATCH, EMBED_DIM), jnp.float32),
        ),
        scratch_types=[
            pltpu.VMEM((_BPW,), jnp.int32),
            pltpu.VMEM((_BPW,), jnp.int32),
            pltpu.VMEM((_CH, 8, EMBED_DIM), jnp.float32),
            pltpu.VMEM((_CH, EMBED_DIM), jnp.float32),
            pltpu.SemaphoreType.DMA,
        ],
    )
    def emb_kernel(e1_hbm, rel_hbm, tab_e_hbm, tab_r_hbm, out_e_hbm,
                   out_r_hbm, idx_e, idx_r, blocks, stage, sem):
        wid = lax.axis_index("s") * _NC + lax.axis_index("c")
        base = wid * _BPW
        pltpu.sync_copy(e1_hbm.at[pl.ds(base, _BPW)], idx_e)
        pltpu.sync_copy(rel_hbm.at[pl.ds(base, _BPW)], idx_r)

        def lookup_table(tab_hbm, idx, out_hbm):
            def chunk_body(k, carry):
                j0 = k * _CH
                for g in range(_CH // _LANES):
                    v_raw = idx[pl.ds(j0 + g * _LANES, _LANES)]
                    v_blk = v_raw >> 3
                    v_sub = v_raw & 7
                    for lane in range(_LANES):
                        j = g * _LANES + lane
                        pltpu.make_async_copy(
                            tab_hbm.at[v_blk[lane], pl.ds(v_sub[lane], 1)],
                            stage.at[pl.ds(j, 1)], sem).start()
                pltpu.make_async_copy(
                    out_hbm.at[pl.ds(base, _CH)], stage, sem).wait()
                pltpu.sync_copy(stage, out_hbm.at[pl.ds(base + j0, _CH)])
                return carry

            lax.fori_loop(0, _NCHUNK, chunk_body, None, unroll=False)

        lookup_table(tab_e_hbm, idx_e, out_e_hbm)
        lookup_table(tab_r_hbm, idx_r, out_r_hbm)

    return emb_kernel


_emb_kernel = _make_kernel()


def kernel(e1, rel, emb_e_weight, emb_rel_weight):
    e1_flat = e1.reshape(BATCH)
    rel_flat = rel.reshape(BATCH)
    tab_e = emb_e_weight.reshape(-1, 8, EMBED_DIM)
    tab_r = emb_rel_weight.reshape(-1, 8, EMBED_DIM)
    return _emb_kernel(e1_flat, rel_flat, tab_e, tab_r)
